# baseline (speedup 1.0000x reference)
import functools

import jax
import jax.numpy as jnp
from jax.experimental import pallas as pl

_NUM_EXPERTS = 16
_BLOCK_TOKENS = 1024


def _router_block(hs_ref, w1_ref, b1_ref, w2_ref, b2_ref, out_ref):
    h = jnp.dot(hs_ref[...], w1_ref[...], preferred_element_type=jnp.float32)
    h = jnp.maximum(h + b1_ref[...], 0.0)
    logits = jax.lax.dot_general(
        w2_ref[...], h, (((0,), (1,)), ((), ())),
        preferred_element_type=jnp.float32,
    )
    logits = logits + b2_ref[...]

    m = jnp.max(logits, axis=0, keepdims=True)
    e = jnp.exp(logits - m)
    s = jnp.sum(e, axis=0, keepdims=True)

    rows = jax.lax.broadcasted_iota(jnp.int32, e.shape, 0)
    idx1 = jnp.min(jnp.where(e == 1.0, rows, _NUM_EXPERTS), axis=0, keepdims=True)
    top1 = rows == idx1
    e_rest = jnp.where(top1, -1.0, e)
    m2 = jnp.max(e_rest, axis=0, keepdims=True)
    idx2 = jnp.min(jnp.where(e_rest == m2, rows, _NUM_EXPERTS), axis=0, keepdims=True)
    keep = top1 | (rows == idx2)

    t = jnp.exp((e - 1.0) / s)
    denom = 1.0 + jnp.exp((m2 - 1.0) / s)
    out = jnp.where(keep, t, 0.0) / denom
    out_ref[...] = out.T


@functools.partial(jax.jit, static_argnames=())
def _router(hs2d, W1, b1, W2, b2):
    n_tokens = hs2d.shape[0]
    d_model = hs2d.shape[1]
    d_hidden = W1.shape[1]
    grid = (n_tokens // _BLOCK_TOKENS,)
    return pl.pallas_call(
        _router_block,
        grid=grid,
        in_specs=[
            pl.BlockSpec((_BLOCK_TOKENS, d_model), lambda i: (i, 0)),
            pl.BlockSpec((d_model, d_hidden), lambda i: (0, 0)),
            pl.BlockSpec((1, d_hidden), lambda i: (0, 0)),
            pl.BlockSpec((d_hidden, _NUM_EXPERTS), lambda i: (0, 0)),
            pl.BlockSpec((_NUM_EXPERTS, 1), lambda i: (0, 0)),
        ],
        out_specs=pl.BlockSpec((_BLOCK_TOKENS, _NUM_EXPERTS), lambda i: (i, 0)),
        out_shape=jax.ShapeDtypeStruct((n_tokens, _NUM_EXPERTS), jnp.float32),
    )(hs2d, W1, b1, W2, b2)


def kernel(hidden_states, W1, b1, W2, b2):
    batch, seq, d_model = hidden_states.shape
    hs2d = hidden_states.reshape(batch * seq, d_model)
    out = _router(hs2d, W1, b1.reshape(1, -1), W2, b2.reshape(-1, 1))
    return out.reshape(batch, seq, _NUM_EXPERTS)


# speedup vs baseline: 1.0237x; 1.0237x over previous
import functools

import jax
import jax.numpy as jnp
from jax.experimental import pallas as pl

_NUM_EXPERTS = 16
_BLOCK_TOKENS = 2048


def _router_block(hs_ref, w1_ref, b1_ref, w2_ref, b2_ref, out_ref):
    h = jnp.dot(hs_ref[...], w1_ref[...], preferred_element_type=jnp.float32)
    h = jnp.maximum(h + b1_ref[...], 0.0)
    logits = jax.lax.dot_general(
        w2_ref[...], h, (((0,), (1,)), ((), ())),
        preferred_element_type=jnp.float32,
    )
    logits = logits + b2_ref[...]

    m = jnp.max(logits, axis=0, keepdims=True)
    e = jnp.exp(logits - m)
    s = jnp.sum(e, axis=0, keepdims=True)

    rows = jax.lax.broadcasted_iota(jnp.int32, e.shape, 0)
    idx1 = jnp.min(jnp.where(e == 1.0, rows, _NUM_EXPERTS), axis=0, keepdims=True)
    top1 = rows == idx1
    e_rest = jnp.where(top1, -1.0, e)
    m2 = jnp.max(e_rest, axis=0, keepdims=True)
    idx2 = jnp.min(jnp.where(e_rest == m2, rows, _NUM_EXPERTS), axis=0, keepdims=True)
    keep = top1 | (rows == idx2)

    t = jnp.exp((e - 1.0) / s)
    denom = 1.0 + jnp.exp((m2 - 1.0) / s)
    out = jnp.where(keep, t, 0.0) / denom
    out_ref[...] = out.T


@functools.partial(jax.jit, static_argnames=())
def _router(hs2d, W1, b1, W2, b2):
    n_tokens = hs2d.shape[0]
    d_model = hs2d.shape[1]
    d_hidden = W1.shape[1]
    grid = (n_tokens // _BLOCK_TOKENS,)
    return pl.pallas_call(
        _router_block,
        grid=grid,
        in_specs=[
            pl.BlockSpec((_BLOCK_TOKENS, d_model), lambda i: (i, 0)),
            pl.BlockSpec((d_model, d_hidden), lambda i: (0, 0)),
            pl.BlockSpec((1, d_hidden), lambda i: (0, 0)),
            pl.BlockSpec((d_hidden, _NUM_EXPERTS), lambda i: (0, 0)),
            pl.BlockSpec((_NUM_EXPERTS, 1), lambda i: (0, 0)),
        ],
        out_specs=pl.BlockSpec((_BLOCK_TOKENS, _NUM_EXPERTS), lambda i: (i, 0)),
        out_shape=jax.ShapeDtypeStruct((n_tokens, _NUM_EXPERTS), jnp.float32),
    )(hs2d, W1, b1, W2, b2)


def kernel(hidden_states, W1, b1, W2, b2):
    batch, seq, d_model = hidden_states.shape
    hs2d = hidden_states.reshape(batch * seq, d_model)
    out = _router(hs2d, W1, b1.reshape(1, -1), W2, b2.reshape(-1, 1))
    return out.reshape(batch, seq, _NUM_EXPERTS)
